# baseline (speedup 1.0000x reference)
import jax
import jax.numpy as jnp
from jax import lax
from jax.experimental import pallas as pl
from jax.experimental.pallas import tpu as pltpu
from jax.experimental.pallas import tpu_sc as plsc

T = 2048
B = 8
D = 500
DP = 512
TOKLEN_WORDS = 4
BAND = 64
NUM_UNITS = B * (T // BAND)
HALF_WORDS = 2 * BAND
NSLICE = (D + 15) // 16


def _sc_body(emb_hbm, pad_hbm, srct_hbm, nq_hbm, out_hbm, len_hbm,
             out_v, g_v, widx_v, srcv, posv, lens_v, tmp_v, nq_v, pos_sh):
    cid = lax.axis_index("c")
    sid = lax.axis_index("s")
    wid = sid * 2 + cid
    lane = lax.iota(jnp.int32, 16)

    @pl.when(sid < B)
    def _():
        pltpu.sync_copy(srct_hbm.at[sid], srcv)
        zero16 = jnp.zeros((16,), jnp.int32)

        def clear(k, _):
            posv[pl.ds(k * 16, 16)] = zero16
            return 0

        lax.fori_loop(0, T // 16, clear, 0)

        cnt = jnp.int32(0)
        for k in range(T // 16):
            xv = srcv[pl.ds(k * 16, 16)]
            maskv = xv != 1
            cntv = plsc.all_reduce_population_count(maskv)
            rowv = k * 16 + lane
            _, sval = plsc.sort_key_val(xv * 65536 + rowv, rowv)
            plsc.store_scatter(posv, [cnt + lane], sval, mask=lane < cntv)
            cnt = cnt + cntv[0]

        pltpu.sync_copy(posv, pos_sh.at[sid])

    plsc.subcore_barrier()

    pltpu.sync_copy(nq_hbm, nq_v)
    nvec = nq_v[pl.ds(0, 16)]
    novec = (nvec + (TOKLEN_WORDS - 1)) // TOKLEN_WORDS
    key = jnp.where(lane < B, lane - novec * 16, (1 << 30) + lane)
    skey, sval = plsc.sort_key_val(key, lane)

    def pick(vec, i):
        plsc.store_scatter(tmp_v, [lane * 16], vec)
        return tmp_v[pl.ds(i * 16, 16)][0]

    col = lax.rem(wid, B)
    src_c = pick(sval, col)
    n_c = pick(nvec, src_c)
    nout_c = (n_c + (TOKLEN_WORDS - 1)) // TOKLEN_WORDS

    @pl.when(wid == 0)
    def _():
        lens_v[pl.ds(0, 16)] = jnp.where(
            lane < B, lax.div(sval - skey, 16), 0)
        pltpu.sync_copy(lens_v, len_hbm)

    def unit(i, _):
        band = lax.div(wid + 32 * i, B)
        r0 = band * BAND
        m = jnp.clip(nout_c - r0, 0, BAND)

        @pl.when(m < BAND)
        def _():
            pltpu.sync_copy(emb_hbm.at[pl.ds(r0, BAND), src_c], out_v)

        for h in range(2):
            s_lo = 32 * h
            s_hi = jnp.minimum(m, s_lo + 32)

            @pl.when(s_hi > s_lo)
            def _():
                pltpu.sync_copy(
                    pos_sh.at[src_c, pl.ds(4 * r0 + HALF_WORDS * h, HALF_WORDS)],
                    widx_v)
                for q in range(HALF_WORDS // 16):
                    widx_v[pl.ds(q * 16, 16)] = (
                        widx_v[pl.ds(q * 16, 16)] * B + src_c)
                pltpu.sync_copy(pad_hbm.at[widx_v], g_v)

                def row(sl, _):
                    s = s_lo + sl
                    nv = n_c - 4 * (r0 + s)
                    zero = jnp.zeros((16,), jnp.float32)
                    for d in range(NSLICE):
                        off = min(d * 16, D - 16)
                        v0 = g_v[4 * sl, pl.ds(off, 16)]
                        v1 = g_v[4 * sl + 1, pl.ds(off, 16)]
                        v2 = g_v[4 * sl + 2, pl.ds(off, 16)]
                        v3 = g_v[4 * sl + 3, pl.ds(off, 16)]
                        acc = v0 + jnp.where(nv > 1, v1, zero)
                        acc = acc + jnp.where(nv > 2, v2, zero)
                        acc = acc + jnp.where(nv > 3, v3, zero)
                        out_v[s, pl.ds(off, 16)] = acc
                    return 0

                lax.fori_loop(0, s_hi - s_lo, row, 0)

        pltpu.sync_copy(out_v, out_hbm.at[pl.ds(r0, BAND), col])
        return 0

    lax.fori_loop(0, NUM_UNITS // 32, unit, 0)


@jax.jit
def _run(embedded, src):
    emb_pad = jnp.pad(embedded.reshape(T * B, D), ((0, 0), (0, DP - D)))
    srcT = src.astype(jnp.int32).T
    nq = jnp.pad(jnp.sum((srcT != 1).astype(jnp.int32), axis=1), (0, 16 - B))

    mesh = plsc.VectorSubcoreMesh(core_axis_name="c", subcore_axis_name="s")
    packed, lens16 = pl.kernel(
        _sc_body,
        mesh=mesh,
        compiler_params=pltpu.CompilerParams(needs_layout_passes=False),
        out_type=(jax.ShapeDtypeStruct((T, B, D), jnp.float32),
                  jax.ShapeDtypeStruct((16,), jnp.int32)),
        scratch_types=[
            pltpu.VMEM((BAND, D), jnp.float32),
            pltpu.VMEM((HALF_WORDS, DP), jnp.float32),
            pltpu.VMEM((HALF_WORDS,), jnp.int32),
            pltpu.VMEM((T,), jnp.int32),
            pltpu.VMEM((T,), jnp.int32),
            pltpu.VMEM((16,), jnp.int32),
            pltpu.VMEM((256,), jnp.int32),
            pltpu.VMEM((16,), jnp.int32),
            pltpu.VMEM_SHARED((B, T), jnp.int32),
        ],
    )(embedded, emb_pad, srcT, nq)

    return packed, lens16[:B]


def kernel(embedded, src, lengths, token_dict):
    return _run(embedded, src)
